# baseline (speedup 1.0000x reference)
import jax
import jax.numpy as jnp
from jax.experimental import pallas as pl
from jax.experimental.pallas import tpu as pltpu


def _prelu(h, a):
    return jnp.where(h >= 0.0, h, a * h)


def _dense(h, w_ref, b_ref):
    return (jnp.dot(h, w_ref[...], preferred_element_type=jnp.float32)
            + b_ref[...])


_NW = 9


def _net_block(*refs):
    x_ref, al_ref = refs[0], refs[1]
    nview = x_ref.shape[0]
    w_hbm = refs[2:2 + nview * _NW]
    biases = refs[2 + nview * _NW:2 + 2 * nview * _NW]
    z_ref, f_ref, r_ref = refs[2 + 2 * nview * _NW:2 + 2 * nview * _NW + 3]
    scratch = refs[2 + 2 * nview * _NW + 3:]
    w_vmem = scratch[:nview * _NW]
    sems = scratch[nview * _NW]

    first = pl.program_id(0) == 0

    @pl.when(first)
    def _start_all():
        for i in range(nview * _NW):
            pltpu.make_async_copy(w_hbm[i], w_vmem[i], sems.at[i]).start()

    def wait(i):
        @pl.when(first)
        def _():
            pltpu.make_async_copy(w_hbm[i], w_vmem[i], sems.at[i]).wait()

    for v in range(nview):
        wb = biases[v * _NW:(v + 1) * _NW]
        wi = v * _NW
        x = x_ref[v]
        al = al_ref[v, 0]

        wait(wi + 0)
        h = _prelu(_dense(x, w_vmem[wi + 0], wb[0]), al[0])
        wait(wi + 1)
        h = _prelu(_dense(h, w_vmem[wi + 1], wb[1]), al[1])
        wait(wi + 2)
        h = _prelu(_dense(h, w_vmem[wi + 2], wb[2]), al[2])
        wait(wi + 3)
        z = _dense(h, w_vmem[wi + 3], wb[3])

        wait(wi + 4)
        g = _prelu(_dense(z, w_vmem[wi + 4], wb[4]), al[3])
        wait(wi + 5)
        g = _prelu(_dense(g, w_vmem[wi + 5], wb[5]), al[4])
        wait(wi + 6)
        g = _prelu(_dense(g, w_vmem[wi + 6], wb[6]), al[5])
        wait(wi + 7)
        r = _dense(g, w_vmem[wi + 7], wb[7])

        wait(wi + 8)
        f = _dense(z, w_vmem[wi + 8], wb[8])

        z_ref[v] = z
        f_ref[v] = f
        r_ref[v] = r


_TILE_CANDIDATES = (400, 256, 200, 128, 80, 64, 40, 32, 16, 8)


def kernel(xs, enc_params, dec_params, proj_params):
    view, n, din = xs.shape
    nlayers = len(enc_params[0])
    tile = next(t for t in _TILE_CANDIDATES if n % t == 0)

    alphas = jnp.stack([
        jnp.concatenate([e[l][2] for l in range(nlayers - 1)]
                        + [d[l][2] for l in range(nlayers - 1)])
        for e, d in zip(enc_params, dec_params)
    ])[:, None, :]

    def const_spec(arr):
        shape = arr.shape
        return pl.BlockSpec(shape, lambda i: (0,) * len(shape))

    weights, bias_ops = [], []
    for v in range(view):
        for (w, b, _a) in enc_params[v]:
            weights.append(w)
            bias_ops.append(b.reshape(1, -1))
        for (w, b, _a) in dec_params[v]:
            weights.append(w)
            bias_ops.append(b.reshape(1, -1))
        pw, pb = proj_params[v]
        weights.append(pw)
        bias_ops.append(pb.reshape(1, -1))

    in_specs = ([pl.BlockSpec((view, tile, din), lambda i: (0, i, 0)),
                 const_spec(alphas)]
                + [pl.BlockSpec(memory_space=pltpu.MemorySpace.HBM)
                   for _ in weights]
                + [const_spec(b) for b in bias_ops])

    feat = enc_params[0][-1][0].shape[-1]
    high = proj_params[0][0].shape[-1]
    out_shape = (
        jax.ShapeDtypeStruct((view, n, feat), xs.dtype),
        jax.ShapeDtypeStruct((view, n, high), xs.dtype),
        jax.ShapeDtypeStruct((view, n, din), xs.dtype),
    )
    out_specs = (
        pl.BlockSpec((view, tile, feat), lambda i: (0, i, 0)),
        pl.BlockSpec((view, tile, high), lambda i: (0, i, 0)),
        pl.BlockSpec((view, tile, din), lambda i: (0, i, 0)),
    )
    scratch_shapes = ([pltpu.VMEM(w.shape, w.dtype) for w in weights]
                      + [pltpu.SemaphoreType.DMA((len(weights),))])

    return pl.pallas_call(
        _net_block,
        grid=(n // tile,),
        in_specs=in_specs,
        out_specs=out_specs,
        out_shape=out_shape,
        scratch_shapes=scratch_shapes,
        compiler_params=pltpu.CompilerParams(
            dimension_semantics=("arbitrary",),
            vmem_limit_bytes=100 * 1024 * 1024,
        ),
    )(xs, alphas, *weights, *bias_ops)


# speedup vs baseline: 1.3465x; 1.3465x over previous
import jax
import jax.numpy as jnp
from jax.experimental import pallas as pl
from jax.experimental.pallas import tpu as pltpu


def _prelu(h, a):
    return jnp.where(h >= 0.0, h, a * h)


def _dense(h, w_ref, b_ref):
    return (jnp.dot(h.astype(w_ref.dtype), w_ref[...],
                    preferred_element_type=jnp.float32)
            + b_ref[...])


def _net_block(*refs):
    x_ref = refs[0]
    al_ref = refs[1]
    z_ref, f_ref, r_ref = refs[-3:]
    nview = x_ref.shape[0]
    per = (len(refs) - 5) // nview
    W = [refs[2 + v * per: 2 + (v + 1) * per] for v in range(nview)]
    al = [al_ref[v, 0] for v in range(nview)]

    h = [x_ref[v] for v in range(nview)]
    for l in range(3):
        h = [_prelu(_dense(h[v], W[v][2 * l], W[v][2 * l + 1]), al[v][l])
             for v in range(nview)]
    z = [_dense(h[v], W[v][6], W[v][7]) for v in range(nview)]

    g = z
    for l in range(3):
        g = [_prelu(_dense(g[v], W[v][8 + 2 * l], W[v][9 + 2 * l]),
                    al[v][3 + l])
             for v in range(nview)]
    r = [_dense(g[v], W[v][14], W[v][15]) for v in range(nview)]
    f = [_dense(z[v], W[v][16], W[v][17]) for v in range(nview)]

    for v in range(nview):
        z_ref[v] = z[v]
        f_ref[v] = f[v]
        r_ref[v] = r[v]


_TILE_CANDIDATES = (400, 256, 200, 128, 80, 64, 40, 32, 16, 8)


def kernel(xs, enc_params, dec_params, proj_params):
    view, n, din = xs.shape
    nlayers = len(enc_params[0])
    tile = next(t for t in _TILE_CANDIDATES if n % t == 0)

    alphas = jnp.stack([
        jnp.concatenate([e[l][2] for l in range(nlayers - 1)]
                        + [d[l][2] for l in range(nlayers - 1)])
        for e, d in zip(enc_params, dec_params)
    ])[:, None, :]

    def const_spec(arr):
        shape = arr.shape
        return pl.BlockSpec(shape, lambda i: (0,) * len(shape))

    operands = []
    in_specs = [pl.BlockSpec((view, tile, din), lambda i: (0, i, 0)),
                const_spec(alphas)]
    for v in range(view):
        ops = []
        for (w, b, _a) in enc_params[v]:
            ops += [w, b.reshape(1, -1)]
        for (w, b, _a) in dec_params[v]:
            ops += [w, b.reshape(1, -1)]
        pw, pb = proj_params[v]
        ops += [pw, pb.reshape(1, -1)]
        operands += ops
        in_specs += [const_spec(o) for o in ops]

    feat = enc_params[0][-1][0].shape[-1]
    high = proj_params[0][0].shape[-1]
    out_shape = (
        jax.ShapeDtypeStruct((view, n, feat), xs.dtype),
        jax.ShapeDtypeStruct((view, n, high), xs.dtype),
        jax.ShapeDtypeStruct((view, n, din), xs.dtype),
    )
    out_specs = (
        pl.BlockSpec((view, tile, feat), lambda i: (0, i, 0)),
        pl.BlockSpec((view, tile, high), lambda i: (0, i, 0)),
        pl.BlockSpec((view, tile, din), lambda i: (0, i, 0)),
    )

    return pl.pallas_call(
        _net_block,
        grid=(n // tile,),
        in_specs=in_specs,
        out_specs=out_specs,
        out_shape=out_shape,
        compiler_params=pltpu.CompilerParams(
            dimension_semantics=("arbitrary",),
            vmem_limit_bytes=100 * 1024 * 1024,
        ),
    )(xs, alphas, *operands)


# speedup vs baseline: 1.3473x; 1.0006x over previous
import jax
import jax.numpy as jnp
from jax.experimental import pallas as pl
from jax.experimental.pallas import tpu as pltpu


def _prelu(h, a):
    return jnp.where(h >= 0.0, h, a * h)


def _dense(h, w_ref, b_ref):
    return (jnp.dot(h, w_ref[...], preferred_element_type=jnp.float32)
            + b_ref[...])


def _net_block(*refs):
    x_ref = refs[0]
    al_ref = refs[1]
    z_ref, f_ref, r_ref = refs[-3:]
    nview = x_ref.shape[0]
    per = (len(refs) - 5) // nview
    W = [refs[2 + v * per: 2 + (v + 1) * per] for v in range(nview)]
    al = [al_ref[v, 0] for v in range(nview)]

    h = [x_ref[v] for v in range(nview)]
    for l in range(3):
        h = [_prelu(_dense(h[v], W[v][2 * l], W[v][2 * l + 1]), al[v][l])
             for v in range(nview)]
    z = [_dense(h[v], W[v][6], W[v][7]) for v in range(nview)]

    g = z
    for l in range(3):
        g = [_prelu(_dense(g[v], W[v][8 + 2 * l], W[v][9 + 2 * l]),
                    al[v][3 + l])
             for v in range(nview)]
    r = [_dense(g[v], W[v][14], W[v][15]) for v in range(nview)]
    f = [_dense(z[v], W[v][16], W[v][17]) for v in range(nview)]

    for v in range(nview):
        z_ref[v] = z[v]
        f_ref[v] = f[v]
        r_ref[v] = r[v]


_TILE_CANDIDATES = (400, 256, 200, 128, 80, 64, 40, 32, 16, 8)


def kernel(xs, enc_params, dec_params, proj_params):
    view, n, din = xs.shape
    nlayers = len(enc_params[0])
    tile = next(t for t in _TILE_CANDIDATES if n % t == 0)

    alphas = jnp.stack([
        jnp.concatenate([e[l][2] for l in range(nlayers - 1)]
                        + [d[l][2] for l in range(nlayers - 1)])
        for e, d in zip(enc_params, dec_params)
    ])[:, None, :]

    def const_spec(arr):
        shape = arr.shape
        return pl.BlockSpec(shape, lambda i: (0,) * len(shape))

    operands = []
    in_specs = [pl.BlockSpec((view, tile, din), lambda i: (0, i, 0)),
                const_spec(alphas)]
    for v in range(view):
        ops = []
        for (w, b, _a) in enc_params[v]:
            ops += [w, b.reshape(1, -1)]
        for (w, b, _a) in dec_params[v]:
            ops += [w, b.reshape(1, -1)]
        pw, pb = proj_params[v]
        ops += [pw, pb.reshape(1, -1)]
        operands += ops
        in_specs += [const_spec(o) for o in ops]

    feat = enc_params[0][-1][0].shape[-1]
    high = proj_params[0][0].shape[-1]
    out_shape = (
        jax.ShapeDtypeStruct((view, n, feat), xs.dtype),
        jax.ShapeDtypeStruct((view, n, high), xs.dtype),
        jax.ShapeDtypeStruct((view, n, din), xs.dtype),
    )
    out_specs = (
        pl.BlockSpec((view, tile, feat), lambda i: (0, i, 0)),
        pl.BlockSpec((view, tile, high), lambda i: (0, i, 0)),
        pl.BlockSpec((view, tile, din), lambda i: (0, i, 0)),
    )

    return pl.pallas_call(
        _net_block,
        grid=(n // tile,),
        in_specs=in_specs,
        out_specs=out_specs,
        out_shape=out_shape,
        compiler_params=pltpu.CompilerParams(
            dimension_semantics=("arbitrary",),
            vmem_limit_bytes=100 * 1024 * 1024,
        ),
    )(xs, alphas, *operands)
